# baseline (speedup 1.0000x reference)
import functools
import math

import jax
import jax.numpy as jnp
from jax import lax
from jax.experimental import pallas as pl
from jax.experimental.pallas import tpu as pltpu
from jax.experimental.pallas import tpu_sc as plsc

_EMBED = 128
_NGAUSS = 50
_RADIUS = 12.0
_B = 4
_NMAX = 288
_NSLOT = _NMAX * _B
_NATOM = 1024
_LANES = 16
_NCHUNK = _NSLOT // _LANES
_NWORKER = 32
_STEP = _RADIUS / (_NGAUSS - 1)
_COEFF = -0.5 / (_STEP * _STEP)
_RSQRT3 = 1.0 / math.sqrt(3.0)


_SC_TILES = 24
_SC_SLOTS = _NSLOT // _SC_TILES


def _sc_body(pos_hbm, nat_hbm, anum_hbm, srct_hbm, dstt_hbm,
             x_out, y_out, z_out, v_out, semb_out, demb_out,
             pos_v, anum_v, nat_v, starts_v, xs_v, ys_v, zs_v, vs_v, idx_v,
             rows_v, rows2_v, sem_in, sem_in2, semg1, semg2, sem_out):
    wid = lax.axis_index("s") * 2 + lax.axis_index("c")

    @pl.when(wid < _SC_TILES)
    def _work():
        base = wid * _SC_SLOTS
        c_pos = pltpu.async_copy(pos_hbm, pos_v, sem_in)
        c_anum = pltpu.async_copy(anum_hbm, anum_v, sem_in2)
        pltpu.sync_copy(nat_hbm, nat_v)
        lane = lax.iota(jnp.int32, _LANES)
        starts = jnp.zeros((_LANES,), jnp.int32)
        for k in (1, 2, 3):
            shifted = plsc.load_gather(nat_v, [jnp.maximum(lane - k, 0)])
            starts = starts + jnp.where(lane >= k, shifted, 0)
        starts_v[...] = starts

        def route(grp):
            s = base + grp * _LANES + lane
            i = lax.shift_right_logical(s, 2)
            b = jnp.bitwise_and(s, 3)
            natb = plsc.load_gather(nat_v, [b])
            stb = plsc.load_gather(starts_v, [b])
            valid = i < natb
            a = jnp.where(valid, stb + i, 0)
            return valid, a

        c_anum.wait()
        routed = []
        for grp in range(_SC_SLOTS // _LANES):
            valid, a = route(grp)
            routed.append((valid, a))
            anum_s = plsc.load_gather(anum_v, [a])
            idx_v[pl.ds(grp * _LANES, _LANES)] = jnp.where(valid, anum_s, 0)
        g1 = pltpu.async_copy(srct_hbm.at[idx_v], rows_v, semg1)
        g2 = pltpu.async_copy(dstt_hbm.at[idx_v], rows2_v, semg2)

        c_pos.wait()
        for grp, (valid, a) in enumerate(routed):
            gsl = pl.ds(grp * _LANES, _LANES)
            a3 = a * 3
            for (col, st_ref) in ((0, xs_v), (1, ys_v), (2, zs_v)):
                coord = plsc.load_gather(pos_v, [a3 + col])
                st_ref[gsl] = jnp.where(valid, coord, 0.0)
            vs_v[gsl] = jnp.where(valid, 1.0, 0.0).astype(jnp.float32)

        osl = pl.ds(base, _SC_SLOTS)
        outs = [
            pltpu.async_copy(xs_v, x_out.at[osl], sem_out),
            pltpu.async_copy(ys_v, y_out.at[osl], sem_out),
            pltpu.async_copy(zs_v, z_out.at[osl], sem_out),
            pltpu.async_copy(vs_v, v_out.at[osl], sem_out),
        ]
        g1.wait()
        outs.append(pltpu.async_copy(rows_v, semb_out.at[osl], sem_out))
        g2.wait()
        outs.append(pltpu.async_copy(rows2_v, demb_out.at[osl], sem_out))
        for cp in outs:
            cp.wait()


@functools.cache
def _sc_build_fn():
    return pl.kernel(
        _sc_body,
        out_type=[
            jax.ShapeDtypeStruct((_NSLOT,), jnp.float32),
            jax.ShapeDtypeStruct((_NSLOT,), jnp.float32),
            jax.ShapeDtypeStruct((_NSLOT,), jnp.float32),
            jax.ShapeDtypeStruct((_NSLOT,), jnp.float32),
            jax.ShapeDtypeStruct((_NSLOT, _EMBED), jnp.float32),
            jax.ShapeDtypeStruct((_NSLOT, _EMBED), jnp.float32),
        ],
        mesh=plsc.VectorSubcoreMesh(
            core_axis_name="c", subcore_axis_name="s"),
        compiler_params=pltpu.CompilerParams(needs_layout_passes=False),
        scratch_types=[
            pltpu.VMEM((_NATOM * 3,), jnp.float32),
            pltpu.VMEM((_NATOM,), jnp.int32),
            pltpu.VMEM((_LANES,), jnp.int32),
            pltpu.VMEM((_LANES,), jnp.int32),
            pltpu.VMEM((_SC_SLOTS,), jnp.float32),
            pltpu.VMEM((_SC_SLOTS,), jnp.float32),
            pltpu.VMEM((_SC_SLOTS,), jnp.float32),
            pltpu.VMEM((_SC_SLOTS,), jnp.float32),
            pltpu.VMEM((_SC_SLOTS,), jnp.int32),
            pltpu.VMEM((_SC_SLOTS, _EMBED), jnp.float32),
            pltpu.VMEM((_SC_SLOTS, _EMBED), jnp.float32),
            pltpu.SemaphoreType.DMA,
            pltpu.SemaphoreType.DMA,
            pltpu.SemaphoreType.DMA,
            pltpu.SemaphoreType.DMA,
            pltpu.SemaphoreType.DMA,
        ],
    )


_BI = 32


def _tc_body(xf, yf, zf, pmv, xs, ys, zs, pms, src3, dstf, rbfw, rbfb,
             feat_o, dist_o, hx_o, hy_o, hz_o, mask_o):
    i0 = pl.program_id(0)
    bl = jnp.bitwise_and(
        lax.broadcasted_iota(jnp.int32, (1, _NSLOT), 1), 3)

    off = lax.broadcasted_iota(
        jnp.int32, (_NGAUSS, _NSLOT), 0).astype(jnp.float32) * _STEP
    r6 = lax.broadcasted_iota(jnp.int32, (5, _NSLOT), 0)
    b6 = jnp.bitwise_and(lax.broadcasted_iota(jnp.int32, (5, _NSLOT), 1), 3)
    extra = jnp.where(r6 == 4, 1.0, jnp.where(r6 == b6, 1.0, 0.0))

    for r in range(_BI):
        def row_from_scalars(sref):
            base = (_BI * i0 + r) * _B
            v0 = sref[0, base]
            v1 = sref[0, base + 1]
            v2 = sref[0, base + 2]
            v3 = sref[0, base + 3]
            return jnp.where(
                bl == 0, v0,
                jnp.where(bl == 1, v1, jnp.where(bl == 2, v2, v3)))

        dx = xf[...] - row_from_scalars(xs)
        dy = yf[...] - row_from_scalars(ys)
        dz = zf[...] - row_from_scalars(zs)
        sq = dx * dx + dy * dy + dz * dz
        dist = jnp.sqrt(jnp.maximum(sq, 1e-24))
        dist_o[r] = dist
        inv = 1.0 / jnp.maximum(dist, 1e-12)
        hx_o[r] = dx * inv
        hy_o[r] = dy * inv
        hz_o[r] = dz * inv
        m = pmv[...] * row_from_scalars(pms)
        mask_o[r] = m > 0.5

        delta = dist - off
        g = jnp.exp(_COEFF * (delta * delta))
        lhs = jnp.concatenate([g, extra], axis=0)
        rhs = jnp.concatenate(
            [rbfw[...], src3[r], rbfb[...]], axis=0)
        mm = lax.dot_general(
            lhs, rhs, (((0,), (0,)), ((), ())),
            preferred_element_type=jnp.float32)
        feat_o[r] = (mm + dstf[...]) * _RSQRT3


def _tc_call(xf, yf, zf, pmf, src3, demb, rbf_w, rbf_b2):
    vec_row = pl.BlockSpec((1, _NSLOT), lambda i: (0, 0))
    smem_row = pl.BlockSpec(memory_space=pltpu.SMEM)
    row3 = pl.BlockSpec((_BI, 1, _NSLOT), lambda i: (i, 0, 0))
    shp3 = jax.ShapeDtypeStruct((_NMAX, 1, _NSLOT), jnp.float32)
    return pl.pallas_call(
        _tc_body,
        grid=(_NMAX // _BI,),
        in_specs=[
            vec_row, vec_row, vec_row, vec_row,
            smem_row, smem_row, smem_row, smem_row,
            pl.BlockSpec((_BI, _B, _EMBED), lambda i: (i, 0, 0)),
            pl.BlockSpec((_NSLOT, _EMBED), lambda i: (0, 0)),
            pl.BlockSpec((_NGAUSS, _EMBED), lambda i: (0, 0)),
            pl.BlockSpec((1, _EMBED), lambda i: (0, 0)),
        ],
        out_specs=[
            pl.BlockSpec((_BI, _NSLOT, _EMBED), lambda i: (i, 0, 0)),
            row3, row3, row3, row3, row3,
        ],
        out_shape=[
            jax.ShapeDtypeStruct((_NMAX, _NSLOT, _EMBED), jnp.float32),
            shp3, shp3, shp3, shp3,
            jax.ShapeDtypeStruct((_NMAX, 1, _NSLOT), jnp.bool_),
        ],
        compiler_params=pltpu.CompilerParams(
            dimension_semantics=("arbitrary",)),
    )(xf, yf, zf, pmf, xf, yf, zf, pmf, src3, demb, rbf_w, rbf_b2)


def kernel(pos, natoms, atomic_numbers, src_table, dst_table, rbf_W, rbf_b):
    src_tbl = src_table.at[0].set(0.0)
    dst_tbl = dst_table.at[0].set(0.0)
    nat16 = jnp.zeros((_LANES,), jnp.int32).at[:_B].set(natoms)

    xcol, ycol, zcol, vcol, semb, demb = _sc_build_fn()(
        pos.reshape(-1), nat16, atomic_numbers, src_tbl, dst_tbl)

    xf = xcol.reshape(1, _NSLOT)
    yf = ycol.reshape(1, _NSLOT)
    zf = zcol.reshape(1, _NSLOT)
    pmf = vcol.reshape(1, _NSLOT)
    src3 = semb.reshape(_NMAX, _B, _EMBED)
    rbf_b2 = rbf_b.reshape(1, _EMBED)

    feat, dist, hx, hy, hz, mrow = _tc_call(
        xf, yf, zf, pmf, src3, demb, rbf_W, rbf_b2)

    padded_features = feat.reshape(_NMAX, _NMAX, _B, _EMBED)
    mask = mrow.reshape(_NMAX, _NMAX, _B)
    padded_mask = vcol.reshape(_NMAX, _B).T > 0.5
    dist_out = dist.reshape(_NMAX, _NMAX, _B)
    vec_hat = jnp.stack(
        [hx.reshape(_NMAX, _NMAX, _B),
         hy.reshape(_NMAX, _NMAX, _B),
         hz.reshape(_NMAX, _NMAX, _B)], axis=-1)
    return padded_features, mask, padded_mask, dist_out, vec_hat


# speedup vs baseline: 1.0049x; 1.0049x over previous
import functools
import math

import jax
import jax.numpy as jnp
from jax import lax
from jax.experimental import pallas as pl
from jax.experimental.pallas import tpu as pltpu
from jax.experimental.pallas import tpu_sc as plsc

_EMBED = 128
_NGAUSS = 50
_RADIUS = 12.0
_B = 4
_NMAX = 288
_NSLOT = _NMAX * _B
_NATOM = 1024
_LANES = 16
_NCHUNK = _NSLOT // _LANES
_NWORKER = 32
_STEP = _RADIUS / (_NGAUSS - 1)
_COEFF = -0.5 / (_STEP * _STEP)
_RSQRT3 = 1.0 / math.sqrt(3.0)


_SC_TILES = 24
_SC_SLOTS = _NSLOT // _SC_TILES


def _sc_body(pos_hbm, nat_hbm, anum_hbm, srct_hbm, dstt_hbm,
             x_out, y_out, z_out, v_out, semb_out, demb_out,
             pos_v, anum_v, nat_v, starts_v, xs_v, ys_v, zs_v, vs_v, idx_v,
             rows_v, rows2_v, sem_in, sem_in2, semg1, semg2, sem_out):
    wid = lax.axis_index("s") * 2 + lax.axis_index("c")

    @pl.when(wid < _SC_TILES)
    def _work():
        base = wid * _SC_SLOTS
        c_pos = pltpu.async_copy(pos_hbm, pos_v, sem_in)
        c_anum = pltpu.async_copy(anum_hbm, anum_v, sem_in2)
        pltpu.sync_copy(nat_hbm, nat_v)
        lane = lax.iota(jnp.int32, _LANES)
        starts = jnp.zeros((_LANES,), jnp.int32)
        for k in (1, 2, 3):
            shifted = plsc.load_gather(nat_v, [jnp.maximum(lane - k, 0)])
            starts = starts + jnp.where(lane >= k, shifted, 0)
        starts_v[...] = starts

        def route(grp):
            s = base + grp * _LANES + lane
            i = lax.shift_right_logical(s, 2)
            b = jnp.bitwise_and(s, 3)
            natb = plsc.load_gather(nat_v, [b])
            stb = plsc.load_gather(starts_v, [b])
            valid = i < natb
            a = jnp.where(valid, stb + i, 0)
            return valid, a

        c_anum.wait()
        routed = []
        for grp in range(_SC_SLOTS // _LANES):
            valid, a = route(grp)
            routed.append((valid, a))
            anum_s = plsc.load_gather(anum_v, [a])
            idx_v[pl.ds(grp * _LANES, _LANES)] = jnp.where(valid, anum_s, 0)
        g1 = pltpu.async_copy(srct_hbm.at[idx_v], rows_v, semg1)
        g2 = pltpu.async_copy(dstt_hbm.at[idx_v], rows2_v, semg2)

        c_pos.wait()
        for grp, (valid, a) in enumerate(routed):
            gsl = pl.ds(grp * _LANES, _LANES)
            a3 = a * 3
            for (col, st_ref) in ((0, xs_v), (1, ys_v), (2, zs_v)):
                coord = plsc.load_gather(pos_v, [a3 + col])
                st_ref[gsl] = jnp.where(valid, coord, 0.0)
            vs_v[gsl] = jnp.where(valid, 1.0, 0.0).astype(jnp.float32)

        osl = pl.ds(base, _SC_SLOTS)
        outs = [
            pltpu.async_copy(xs_v, x_out.at[osl], sem_out),
            pltpu.async_copy(ys_v, y_out.at[osl], sem_out),
            pltpu.async_copy(zs_v, z_out.at[osl], sem_out),
            pltpu.async_copy(vs_v, v_out.at[osl], sem_out),
        ]
        g1.wait()
        outs.append(pltpu.async_copy(rows_v, semb_out.at[osl], sem_out))
        g2.wait()
        outs.append(pltpu.async_copy(rows2_v, demb_out.at[osl], sem_out))
        for cp in outs:
            cp.wait()


@functools.cache
def _sc_build_fn():
    return pl.kernel(
        _sc_body,
        out_type=[
            jax.ShapeDtypeStruct((_NSLOT,), jnp.float32),
            jax.ShapeDtypeStruct((_NSLOT,), jnp.float32),
            jax.ShapeDtypeStruct((_NSLOT,), jnp.float32),
            jax.ShapeDtypeStruct((_NSLOT,), jnp.float32),
            jax.ShapeDtypeStruct((_NSLOT, _EMBED), jnp.float32),
            jax.ShapeDtypeStruct((_NSLOT, _EMBED), jnp.float32),
        ],
        mesh=plsc.VectorSubcoreMesh(
            core_axis_name="c", subcore_axis_name="s"),
        compiler_params=pltpu.CompilerParams(needs_layout_passes=False),
        scratch_types=[
            pltpu.VMEM((_NATOM * 3,), jnp.float32),
            pltpu.VMEM((_NATOM,), jnp.int32),
            pltpu.VMEM((_LANES,), jnp.int32),
            pltpu.VMEM((_LANES,), jnp.int32),
            pltpu.VMEM((_SC_SLOTS,), jnp.float32),
            pltpu.VMEM((_SC_SLOTS,), jnp.float32),
            pltpu.VMEM((_SC_SLOTS,), jnp.float32),
            pltpu.VMEM((_SC_SLOTS,), jnp.float32),
            pltpu.VMEM((_SC_SLOTS,), jnp.int32),
            pltpu.VMEM((_SC_SLOTS, _EMBED), jnp.float32),
            pltpu.VMEM((_SC_SLOTS, _EMBED), jnp.float32),
            pltpu.SemaphoreType.DMA,
            pltpu.SemaphoreType.DMA,
            pltpu.SemaphoreType.DMA,
            pltpu.SemaphoreType.DMA,
            pltpu.SemaphoreType.DMA,
        ],
    )


_BI = 24


def _tc_body(xf, yf, zf, pmv, xs, ys, zs, pms, src3, dstf, rbfw, rbfb,
             feat_o, dist_o, hx_o, hy_o, hz_o, mask_o):
    i0 = pl.program_id(0)
    bl = jnp.bitwise_and(
        lax.broadcasted_iota(jnp.int32, (1, _NSLOT), 1), 3)

    off = lax.broadcasted_iota(
        jnp.int32, (_NGAUSS, _NSLOT), 0).astype(jnp.float32) * _STEP
    r6 = lax.broadcasted_iota(jnp.int32, (5, _NSLOT), 0)
    b6 = jnp.bitwise_and(lax.broadcasted_iota(jnp.int32, (5, _NSLOT), 1), 3)
    extra = jnp.where(r6 == 4, 1.0, jnp.where(r6 == b6, 1.0, 0.0))

    for r in range(_BI):
        def row_from_scalars(sref):
            base = (_BI * i0 + r) * _B
            v0 = sref[0, base]
            v1 = sref[0, base + 1]
            v2 = sref[0, base + 2]
            v3 = sref[0, base + 3]
            return jnp.where(
                bl == 0, v0,
                jnp.where(bl == 1, v1, jnp.where(bl == 2, v2, v3)))

        dx = xf[...] - row_from_scalars(xs)
        dy = yf[...] - row_from_scalars(ys)
        dz = zf[...] - row_from_scalars(zs)
        sq = dx * dx + dy * dy + dz * dz
        dist = jnp.sqrt(jnp.maximum(sq, 1e-24))
        dist_o[r] = dist
        inv = 1.0 / jnp.maximum(dist, 1e-12)
        hx_o[r] = dx * inv
        hy_o[r] = dy * inv
        hz_o[r] = dz * inv
        m = pmv[...] * row_from_scalars(pms)
        mask_o[r] = m > 0.5

        delta = dist - off
        g = jnp.exp(_COEFF * (delta * delta))
        lhs = jnp.concatenate([g, extra], axis=0)
        rhs = jnp.concatenate(
            [rbfw[...], src3[r], rbfb[...]], axis=0)
        mm = lax.dot_general(
            lhs, rhs, (((0,), (0,)), ((), ())),
            preferred_element_type=jnp.float32)
        feat_o[r] = (mm + dstf[...]) * _RSQRT3


def _tc_call(xf, yf, zf, pmf, src3, demb, rbf_w, rbf_b2):
    vec_row = pl.BlockSpec((1, _NSLOT), lambda i: (0, 0))
    smem_row = pl.BlockSpec(memory_space=pltpu.SMEM)
    row3 = pl.BlockSpec((_BI, 1, _NSLOT), lambda i: (i, 0, 0))
    shp3 = jax.ShapeDtypeStruct((_NMAX, 1, _NSLOT), jnp.float32)
    return pl.pallas_call(
        _tc_body,
        grid=(_NMAX // _BI,),
        in_specs=[
            vec_row, vec_row, vec_row, vec_row,
            smem_row, smem_row, smem_row, smem_row,
            pl.BlockSpec((_BI, _B, _EMBED), lambda i: (i, 0, 0)),
            pl.BlockSpec((_NSLOT, _EMBED), lambda i: (0, 0)),
            pl.BlockSpec((_NGAUSS, _EMBED), lambda i: (0, 0)),
            pl.BlockSpec((1, _EMBED), lambda i: (0, 0)),
        ],
        out_specs=[
            pl.BlockSpec((_BI, _NSLOT, _EMBED), lambda i: (i, 0, 0)),
            row3, row3, row3, row3, row3,
        ],
        out_shape=[
            jax.ShapeDtypeStruct((_NMAX, _NSLOT, _EMBED), jnp.float32),
            shp3, shp3, shp3, shp3,
            jax.ShapeDtypeStruct((_NMAX, 1, _NSLOT), jnp.bool_),
        ],
        compiler_params=pltpu.CompilerParams(
            dimension_semantics=("arbitrary",)),
    )(xf, yf, zf, pmf, xf, yf, zf, pmf, src3, demb, rbf_w, rbf_b2)


def kernel(pos, natoms, atomic_numbers, src_table, dst_table, rbf_W, rbf_b):
    src_tbl = src_table.at[0].set(0.0)
    dst_tbl = dst_table.at[0].set(0.0)
    nat16 = jnp.zeros((_LANES,), jnp.int32).at[:_B].set(natoms)

    xcol, ycol, zcol, vcol, semb, demb = _sc_build_fn()(
        pos.reshape(-1), nat16, atomic_numbers, src_tbl, dst_tbl)

    xf = xcol.reshape(1, _NSLOT)
    yf = ycol.reshape(1, _NSLOT)
    zf = zcol.reshape(1, _NSLOT)
    pmf = vcol.reshape(1, _NSLOT)
    src3 = semb.reshape(_NMAX, _B, _EMBED)
    rbf_b2 = rbf_b.reshape(1, _EMBED)

    feat, dist, hx, hy, hz, mrow = _tc_call(
        xf, yf, zf, pmf, src3, demb, rbf_W, rbf_b2)

    padded_features = feat.reshape(_NMAX, _NMAX, _B, _EMBED)
    mask = mrow.reshape(_NMAX, _NMAX, _B)
    padded_mask = vcol.reshape(_NMAX, _B).T > 0.5
    dist_out = dist.reshape(_NMAX, _NMAX, _B)
    vec_hat = jnp.stack(
        [hx.reshape(_NMAX, _NMAX, _B),
         hy.reshape(_NMAX, _NMAX, _B),
         hz.reshape(_NMAX, _NMAX, _B)], axis=-1)
    return padded_features, mask, padded_mask, dist_out, vec_hat
